# baseline (speedup 1.0000x reference)
import functools

import jax
import jax.numpy as jnp
from jax import lax
from jax.experimental import pallas as pl
from jax.experimental.pallas import tpu as pltpu
from jax.experimental.pallas import tpu_sc as plsc

_LANES = 16
_NC = 2


def kernel(features, query_letters):
    n, h = features.shape
    win = n - _LANES
    mesh = plsc.VectorSubcoreMesh(core_axis_name="c", subcore_axis_name="s")

    @functools.partial(
        pl.kernel,
        mesh=mesh,
        out_type=jax.ShapeDtypeStruct((n * h,), features.dtype),
        scratch_types=[
            pltpu.VMEM((n,), jnp.int32),
            pltpu.VMEM((h,), features.dtype),
            pltpu.SemaphoreType.DMA,
            pltpu.SemaphoreType.DMA,
        ],
    )
    def scatter_rows(feat_hbm, q_hbm, out_hbm, q_v, row0, s0, s3):
        wid = lax.axis_index("s") * _NC + lax.axis_index("c")

        @pl.when(wid < n)
        def _():
            in0 = pltpu.make_async_copy(feat_hbm.at[wid], row0, s0)
            in0.start()
            pltpu.sync_copy(q_hbm, q_v)
            base0 = jnp.minimum(wid, win)
            vec = q_v[pl.ds(base0, _LANES)]
            lane = wid - base0
            q0 = vec[0]
            for l in range(1, _LANES):
                q0 = jnp.where(lane == l, vec[l], q0)
            off0 = pl.multiple_of(q0 * h, h)
            in0.wait()
            out0 = pltpu.make_async_copy(row0, out_hbm.at[pl.ds(off0, h)], s3)
            out0.start()
            out0.wait()

    return scatter_rows(features, query_letters.astype(jnp.int32))
